# baseline (speedup 1.0000x reference)
import functools

import jax
import jax.numpy as jnp
import numpy as np
from jax.experimental import pallas as pl
from jax.experimental.pallas import tpu as pltpu

_NUM_SCAT = 24
_GROUP = 8
_LANES = 128
_GROUPS_PER_ROW = _LANES // _GROUP
_PVEC = np.array([0.5 + 0.04 * sc for sc in range(_NUM_SCAT)], dtype=np.float32)

_SEL = np.zeros((_LANES, _GROUPS_PER_ROW), dtype=np.float32)
for _g in range(_GROUPS_PER_ROW):
    _SEL[8 * _g, _g] = 1.0


def _body(x_ref, sel_ref, scale_ref, o_ref):
    m = x_ref[...]
    m = jnp.maximum(m, pltpu.roll(m, _LANES - 4, 1))
    m = jnp.maximum(m, pltpu.roll(m, _LANES - 2, 1))
    m = jnp.maximum(m, pltpu.roll(m, _LANES - 1, 1))
    g = jnp.dot(m, sel_ref[...], preferred_element_type=jnp.float32,
                precision=jax.lax.Precision.HIGHEST)
    o_ref[...] = g * scale_ref[...]


@functools.partial(jax.jit, static_argnums=(1,))
def _run(xr, block_rows):
    rows = xr.shape[0]
    grid = rows // block_rows
    ridx = np.arange(block_rows)[:, None]
    gidx = np.arange(_GROUPS_PER_ROW)[None, :]
    scale = _PVEC[(16 * ridx + gidx) % _NUM_SCAT]
    return pl.pallas_call(
        _body,
        grid=(grid,),
        in_specs=[
            pl.BlockSpec((block_rows, _LANES), lambda i: (i, 0)),
            pl.BlockSpec((_LANES, _GROUPS_PER_ROW), lambda i: (0, 0)),
            pl.BlockSpec((block_rows, _GROUPS_PER_ROW), lambda i: (0, 0)),
        ],
        out_specs=pl.BlockSpec((block_rows, _GROUPS_PER_ROW), lambda i: (i, 0)),
        out_shape=jax.ShapeDtypeStruct((rows, _GROUPS_PER_ROW), jnp.float32),
    )(xr, jnp.asarray(_SEL), jnp.asarray(scale))


def kernel(x):
    b, h, w, c = x.shape
    rows = b * h * w * c // _LANES
    xr = x.reshape(rows, _LANES)
    out = _run(xr, 3072)
    return out.reshape(b, h, w, _NUM_SCAT)


# speedup vs baseline: 1.1801x; 1.1801x over previous
import functools

import jax
import jax.numpy as jnp
import numpy as np
from jax.experimental import pallas as pl
from jax.experimental.pallas import tpu as pltpu

_NUM_SCAT = 24
_GROUP = 8
_C = _NUM_SCAT * _GROUP
_PVEC = np.array([0.5 + 0.04 * sc for sc in range(_NUM_SCAT)], dtype=np.float32)

_SEL = np.zeros((_C, _NUM_SCAT), dtype=np.float32)
for _sc in range(_NUM_SCAT):
    _SEL[8 * _sc, _sc] = _PVEC[_sc]


def _body(x_ref, sel_ref, o_ref):
    m = x_ref[...]
    m = jnp.maximum(m, pltpu.roll(m, _C - 4, 1))
    m = jnp.maximum(m, pltpu.roll(m, _C - 2, 1))
    m = jnp.maximum(m, pltpu.roll(m, _C - 1, 1))
    o_ref[...] = jnp.dot(m, sel_ref[...], preferred_element_type=jnp.float32,
                         precision=jax.lax.Precision.HIGHEST)


@functools.partial(jax.jit, static_argnums=(1,))
def _run(xr, block_rows):
    rows = xr.shape[0]
    grid = rows // block_rows
    return pl.pallas_call(
        _body,
        grid=(grid,),
        in_specs=[
            pl.BlockSpec((block_rows, _C), lambda i: (i, 0)),
            pl.BlockSpec((_C, _NUM_SCAT), lambda i: (0, 0)),
        ],
        out_specs=pl.BlockSpec((block_rows, _NUM_SCAT), lambda i: (i, 0)),
        out_shape=jax.ShapeDtypeStruct((rows, _NUM_SCAT), jnp.float32),
    )(xr, jnp.asarray(_SEL))


def kernel(x):
    b, h, w, c = x.shape
    rows = b * h * w
    xr = x.reshape(rows, c)
    out = _run(xr, 2048)
    return out.reshape(b, h, w, _NUM_SCAT)


# speedup vs baseline: 2.6449x; 2.2413x over previous
import functools

import jax
import jax.numpy as jnp
import numpy as np
from jax.experimental import pallas as pl
from jax.experimental.pallas import tpu as pltpu

_NUM_SCAT = 24
_GROUP = 8
_C = _NUM_SCAT * _GROUP
_PVEC = np.array([0.5 + 0.04 * sc for sc in range(_NUM_SCAT)], dtype=np.float32)

_SEL = np.zeros((_C, _NUM_SCAT), dtype=np.float32)
for _sc in range(_NUM_SCAT):
    _SEL[8 * _sc, _sc] = _PVEC[_sc]


def _body(x_ref, sel_ref, o_ref):
    hb = x_ref.shape[1]
    w = x_ref.shape[2]
    m = x_ref[...].reshape(hb * w, _C)
    m = jnp.maximum(m, pltpu.roll(m, _C - 4, 1))
    m = jnp.maximum(m, pltpu.roll(m, _C - 2, 1))
    m = jnp.maximum(m, pltpu.roll(m, _C - 1, 1))
    g = jnp.dot(m, sel_ref[...], preferred_element_type=jnp.float32)
    o_ref[...] = g.reshape(1, hb, w, _NUM_SCAT)


@functools.partial(jax.jit, static_argnums=(1,))
def _run(x, hb):
    b, h, w, c = x.shape
    return pl.pallas_call(
        _body,
        grid=(b, h // hb),
        in_specs=[
            pl.BlockSpec((1, hb, w, c), lambda i, j: (i, j, 0, 0)),
            pl.BlockSpec((_C, _NUM_SCAT), lambda i, j: (0, 0)),
        ],
        out_specs=pl.BlockSpec((1, hb, w, _NUM_SCAT), lambda i, j: (i, j, 0, 0)),
        out_shape=jax.ShapeDtypeStruct((b, h, w, _NUM_SCAT), jnp.float32),
    )(x, jnp.asarray(_SEL))


def kernel(x):
    return _run(x, 8)


# speedup vs baseline: 3.4340x; 1.2984x over previous
import functools

import jax
import jax.numpy as jnp
import numpy as np
from jax.experimental import pallas as pl

_NUM_SCAT = 24
_C = 192
_PVEC = np.array([0.5 + 0.04 * sc for sc in range(_NUM_SCAT)], dtype=np.float32)


def _level_mat(n_in, n_pairs, scale=None):
    w = np.zeros((n_in, 256), dtype=np.float32)
    for j in range(n_pairs):
        s = 1.0 if scale is None else scale[j]
        w[2 * j, j] = s
        w[2 * j, 128 + j] = -s
        w[2 * j + 1, 128 + j] = s
    return w


_W1 = _level_mat(_C, 96)
_W2 = _level_mat(128, 48)
_W3 = _level_mat(128, 24, _PVEC)


def _mm_max(v, w_ref):
    t = jnp.dot(v, w_ref[...], preferred_element_type=jnp.float32)
    return t[:, :128] + jax.nn.relu(t[:, 128:])


def _body(x_ref, w1_ref, w2_ref, w3_ref, o_ref):
    hb = x_ref.shape[1]
    w = x_ref.shape[2]
    m = x_ref[...].reshape(hb * w, _C)
    s1 = _mm_max(m, w1_ref)
    s2 = _mm_max(s1, w2_ref)
    s3 = _mm_max(s2, w3_ref)
    o_ref[...] = s3[:, :_NUM_SCAT].reshape(1, hb, w, _NUM_SCAT)


@functools.partial(jax.jit, static_argnums=(1,))
def _run(x, hb):
    b, h, w, c = x.shape
    return pl.pallas_call(
        _body,
        grid=(b, h // hb),
        in_specs=[
            pl.BlockSpec((1, hb, w, c), lambda i, j: (i, j, 0, 0)),
            pl.BlockSpec(_W1.shape, lambda i, j: (0, 0)),
            pl.BlockSpec(_W2.shape, lambda i, j: (0, 0)),
            pl.BlockSpec(_W3.shape, lambda i, j: (0, 0)),
        ],
        out_specs=pl.BlockSpec((1, hb, w, _NUM_SCAT), lambda i, j: (i, j, 0, 0)),
        out_shape=jax.ShapeDtypeStruct((b, h, w, _NUM_SCAT), jnp.float32),
    )(x, jnp.asarray(_W1), jnp.asarray(_W2), jnp.asarray(_W3))


def kernel(x):
    return _run(x, 16)
